# baseline (speedup 1.0000x reference)
import jax
import jax.numpy as jnp
from jax.experimental import pallas as pl
from jax.experimental.pallas import tpu as pltpu

B, L, D = 16, 4096, 256
TL = 2048
NB = L // TL
N = B * L


def _in_index_map(b, j, seq_ref):
    last_valid = jnp.maximum((seq_ref[b] + TL - 1) // TL - 1, 0)
    return (b, jnp.minimum(j, last_valid), 0)


def _stats_kernel(seq_ref, x_ref, y_ref, stats_ref, acc_ref):
    b = pl.program_id(0)
    j = pl.program_id(1)

    @pl.when((b == 0) & (j == 0))
    def _():
        acc_ref[...] = jnp.zeros_like(acc_ref)

    start = j * TL
    slen = seq_ref[b]
    col0 = jax.lax.broadcasted_iota(jnp.int32, (TL, D), 1) == 0

    def accumulate(xm, ym):
        acc_ref[0] += jnp.sum(xm, axis=0, keepdims=True)
        acc_ref[1] += jnp.sum(xm * xm, axis=0, keepdims=True)
        acc_ref[2] += jnp.sum(ym, axis=0, keepdims=True)
        acc_ref[3] += jnp.sum(ym * ym, axis=0, keepdims=True)

    @pl.when(start + TL <= slen)
    def _():
        xb = x_ref[0]
        yb = y_ref[0]
        ym = jnp.where(col0, xb - yb, yb)
        accumulate(xb, ym)

    @pl.when((start < slen) & (start + TL > slen))
    def _():
        xb = x_ref[0]
        yb = y_ref[0]
        rows = jax.lax.broadcasted_iota(jnp.int32, (TL, 1), 0) + start
        valid = rows < slen
        xm = jnp.where(valid, xb, 0.0)
        ym = jnp.where(valid, yb, 0.0)
        ym = jnp.where(col0, xm - ym, ym)
        accumulate(xm, ym)

    @pl.when((b == B - 1) & (j == NB - 1))
    def _():
        inv_n = 1.0 / N
        inv_nm1 = 1.0 / (N - 1)
        x_loc = acc_ref[0] * inv_n
        y_loc = acc_ref[2] * inv_n
        x_var = (acc_ref[1] - N * x_loc * x_loc) * inv_nm1
        y_var = (acc_ref[3] - N * y_loc * y_loc) * inv_nm1
        x_rs = jax.lax.rsqrt(x_var)
        y_rs = jax.lax.rsqrt(y_var)
        stats_ref[...] = jnp.concatenate(
            [x_rs, -x_loc * x_rs, y_rs, -y_loc * y_rs,
             jnp.zeros((4, D), jnp.float32)], axis=0)


def _norm_kernel(seq_ref, stats_ref, x_ref, y_ref, xo_ref, yo_ref):
    b = pl.program_id(0)
    j = pl.program_id(1)
    start = j * TL
    slen = seq_ref[b]

    x_rs = stats_ref[0:1]
    x_fill = stats_ref[1:2]
    y_rs = stats_ref[2:3]
    y_fill = stats_ref[3:4]
    col0 = jax.lax.broadcasted_iota(jnp.int32, (TL, D), 1) == 0

    @pl.when(start + TL <= slen)
    def _():
        xb = x_ref[0]
        yb = y_ref[0]
        xo_ref[0] = xb * x_rs + x_fill
        ym = jnp.where(col0, xb - yb, yb)
        yo_ref[0] = ym * y_rs + y_fill

    @pl.when((start < slen) & (start + TL > slen))
    def _():
        xb = x_ref[0]
        yb = y_ref[0]
        rows = jax.lax.broadcasted_iota(jnp.int32, (TL, 1), 0) + start
        valid = rows < slen
        xo_ref[0] = jnp.where(valid, xb * x_rs + x_fill,
                              jnp.broadcast_to(x_fill, (TL, D)))
        ym = jnp.where(col0, xb - yb, yb)
        yo_ref[0] = jnp.where(valid, ym * y_rs + y_fill,
                              jnp.broadcast_to(y_fill, (TL, D)))

    @pl.when(start >= slen)
    def _():
        xo_ref[0] = jnp.broadcast_to(x_fill, (TL, D))
        yo_ref[0] = jnp.broadcast_to(y_fill, (TL, D))


def _mask_kernel(seq_ref, m_ref):
    cols = jax.lax.broadcasted_iota(jnp.int32, (1, L), 1)
    for b in range(B):
        m_ref[b:b + 1, :] = (cols < seq_ref[b]).astype(jnp.float32)


def kernel(x, y, seq_len):
    seq32 = seq_len.astype(jnp.int32)

    stats = pl.pallas_call(
        _stats_kernel,
        grid_spec=pltpu.PrefetchScalarGridSpec(
            num_scalar_prefetch=1,
            grid=(B, NB),
            in_specs=[
                pl.BlockSpec((1, TL, D), _in_index_map),
                pl.BlockSpec((1, TL, D), _in_index_map),
            ],
            out_specs=pl.BlockSpec((8, D), lambda b, j, seq_ref: (0, 0)),
            scratch_shapes=[pltpu.VMEM((4, 1, D), jnp.float32)],
        ),
        out_shape=jax.ShapeDtypeStruct((8, D), jnp.float32),
        compiler_params=pltpu.CompilerParams(
            dimension_semantics=("arbitrary", "arbitrary")),
    )(seq32, x, y)

    x_out, y_out = pl.pallas_call(
        _norm_kernel,
        grid_spec=pltpu.PrefetchScalarGridSpec(
            num_scalar_prefetch=1,
            grid=(B, NB),
            in_specs=[
                pl.BlockSpec((8, D), lambda b, j, seq_ref: (0, 0)),
                pl.BlockSpec((1, TL, D), _in_index_map),
                pl.BlockSpec((1, TL, D), _in_index_map),
            ],
            out_specs=[
                pl.BlockSpec((1, TL, D), lambda b, j, seq_ref: (b, j, 0)),
                pl.BlockSpec((1, TL, D), lambda b, j, seq_ref: (b, j, 0)),
            ],
        ),
        out_shape=[
            jax.ShapeDtypeStruct((B, L, D), jnp.float32),
            jax.ShapeDtypeStruct((B, L, D), jnp.float32),
        ],
        compiler_params=pltpu.CompilerParams(
            dimension_semantics=("parallel", "parallel")),
    )(seq32, stats, x, y)

    mask_f = pl.pallas_call(
        _mask_kernel,
        in_specs=[pl.BlockSpec(memory_space=pltpu.SMEM)],
        out_specs=pl.BlockSpec((B, L), lambda: (0, 0)),
        out_shape=jax.ShapeDtypeStruct((B, L), jnp.float32),
    )(seq32)

    mask = mask_f.astype(bool)
    return (x_out, y_out, seq_len, mask)


# speedup vs baseline: 1.2080x; 1.2080x over previous
import jax
import jax.numpy as jnp
from jax.experimental import pallas as pl
from jax.experimental.pallas import tpu as pltpu

B, L, D = 16, 4096, 256
NQ = 4
TQ = L // NQ
N = B * L


def _q_index_map(q):
    def imap(b, seq_ref, tab_ref):
        return (tab_ref[q, b], q, 0)
    return imap


def _finalize(acc_ref, stats_ref):
    inv_n = 1.0 / N
    inv_nm1 = 1.0 / (N - 1)
    x_loc = acc_ref[0] * inv_n
    y_loc = acc_ref[2] * inv_n
    x_var = (acc_ref[1] - N * x_loc * x_loc) * inv_nm1
    y_var = (acc_ref[3] - N * y_loc * y_loc) * inv_nm1
    x_rs = jax.lax.rsqrt(x_var)
    y_rs = jax.lax.rsqrt(y_var)
    stats_ref[...] = jnp.concatenate(
        [x_rs, -x_loc * x_rs, y_rs, -y_loc * y_rs,
         jnp.zeros((4, D), jnp.float32)], axis=0)


def _stats_kernel(seq_ref, tab_ref, *refs):
    x_refs = refs[0:NQ]
    y_refs = refs[NQ:2 * NQ]
    stats_ref = refs[2 * NQ]
    acc_ref = refs[2 * NQ + 1]
    b = pl.program_id(0)

    @pl.when(b == 0)
    def _():
        acc_ref[...] = jnp.zeros_like(acc_ref)

    slen = seq_ref[b]
    col0 = jax.lax.broadcasted_iota(jnp.int32, (1, D), 1) == 0

    def accumulate(xm, ym, d):
        acc_ref[0] += jnp.sum(xm, axis=0, keepdims=True)
        acc_ref[1] += jnp.sum(xm * xm, axis=0, keepdims=True)
        s_y = jnp.sum(ym, axis=0, keepdims=True)
        ss_y = jnp.sum(ym * ym, axis=0, keepdims=True)
        s_d = jnp.sum(d, axis=0, keepdims=True)
        ss_d = jnp.sum(d * d, axis=0, keepdims=True)
        acc_ref[2] += jnp.where(col0, s_d, s_y)
        acc_ref[3] += jnp.where(col0, ss_d, ss_y)

    for q in range(NQ):
        start = q * TQ

        @pl.when(start + TQ <= slen)
        def _(q=q, start=start):
            xb = x_refs[q][0]
            yb = y_refs[q][0]
            d = xb[:, 0:1] - yb[:, 0:1]
            accumulate(xb, yb, d)

        @pl.when((start < slen) & (start + TQ > slen))
        def _(q=q, start=start):
            xb = x_refs[q][0]
            yb = y_refs[q][0]
            rows = jax.lax.broadcasted_iota(jnp.int32, (TQ, 1), 0) + start
            valid = rows < slen
            xm = jnp.where(valid, xb, 0.0)
            ym = jnp.where(valid, yb, 0.0)
            d = xm[:, 0:1] - ym[:, 0:1]
            accumulate(xm, ym, d)

    @pl.when(b == B - 1)
    def _():
        _finalize(acc_ref, stats_ref)


def _norm_kernel(seq_ref, tab_ref, *refs):
    stats_ref = refs[0]
    x_refs = refs[1:1 + NQ]
    y_refs = refs[1 + NQ:1 + 2 * NQ]
    xo_ref = refs[1 + 2 * NQ]
    yo_ref = refs[2 + 2 * NQ]
    b = pl.program_id(0)
    slen = seq_ref[b]

    x_rs = stats_ref[0:1]
    x_fill = stats_ref[1:2]
    y_rs = stats_ref[2:3]
    y_fill = stats_ref[3:4]
    y_rs0 = stats_ref[2:3, 0:1]
    y_fill0 = stats_ref[3:4, 0:1]

    for q in range(NQ):
        start = q * TQ
        sl = slice(start, start + TQ)

        @pl.when(start + TQ <= slen)
        def _(q=q, sl=sl):
            xb = x_refs[q][0]
            yb = y_refs[q][0]
            xo_ref[0, sl, :] = xb * x_rs + x_fill
            yo_ref[0, sl, :] = yb * y_rs + y_fill
            yo_ref[0, sl, 0:1] = (xb[:, 0:1] - yb[:, 0:1]) * y_rs0 + y_fill0

        @pl.when((start < slen) & (start + TQ > slen))
        def _(q=q, sl=sl, start=start):
            xb = x_refs[q][0]
            yb = y_refs[q][0]
            rows = jax.lax.broadcasted_iota(jnp.int32, (TQ, 1), 0) + start
            valid = rows < slen
            xo_ref[0, sl, :] = jnp.where(
                valid, xb * x_rs + x_fill, jnp.broadcast_to(x_fill, (TQ, D)))
            yo_ref[0, sl, :] = jnp.where(
                valid, yb * y_rs + y_fill, jnp.broadcast_to(y_fill, (TQ, D)))
            d = xb[:, 0:1] - yb[:, 0:1]
            yo_ref[0, sl, 0:1] = jnp.where(
                valid, d * y_rs0 + y_fill0,
                jnp.broadcast_to(y_fill0, (TQ, 1)))

        @pl.when(start >= slen)
        def _(sl=sl):
            xo_ref[0, sl, :] = jnp.broadcast_to(x_fill, (TQ, D))
            yo_ref[0, sl, :] = jnp.broadcast_to(y_fill, (TQ, D))


def _mask_kernel(seq_ref, m_ref):
    cols = jax.lax.broadcasted_iota(jnp.int32, (1, L), 1)
    for b in range(B):
        m_ref[b:b + 1, :] = (cols < seq_ref[b]).astype(jnp.float32)


def kernel(x, y, seq_len):
    seq32 = seq_len.astype(jnp.int32)

    thresh = (jnp.arange(NQ, dtype=jnp.int32) * TQ)[:, None]
    ok = seq32[None, :] > thresh
    idx = jnp.where(ok, jnp.arange(B, dtype=jnp.int32)[None, :], -1)
    tab = jnp.maximum(jax.lax.cummax(idx, axis=1), 0)

    q_in_specs = [pl.BlockSpec((1, TQ, D), _q_index_map(q)) for q in range(NQ)]

    stats = pl.pallas_call(
        _stats_kernel,
        grid_spec=pltpu.PrefetchScalarGridSpec(
            num_scalar_prefetch=2,
            grid=(B,),
            in_specs=q_in_specs + q_in_specs,
            out_specs=pl.BlockSpec((8, D), lambda b, seq_ref, tab_ref: (0, 0)),
            scratch_shapes=[pltpu.VMEM((4, 1, D), jnp.float32)],
        ),
        out_shape=jax.ShapeDtypeStruct((8, D), jnp.float32),
        compiler_params=pltpu.CompilerParams(
            dimension_semantics=("arbitrary",)),
    )(seq32, tab, x, x, x, x, y, y, y, y)

    x_out, y_out = pl.pallas_call(
        _norm_kernel,
        grid_spec=pltpu.PrefetchScalarGridSpec(
            num_scalar_prefetch=2,
            grid=(B,),
            in_specs=[pl.BlockSpec((8, D), lambda b, seq_ref, tab_ref: (0, 0))]
            + q_in_specs + q_in_specs,
            out_specs=[
                pl.BlockSpec((1, L, D), lambda b, seq_ref, tab_ref: (b, 0, 0)),
                pl.BlockSpec((1, L, D), lambda b, seq_ref, tab_ref: (b, 0, 0)),
            ],
        ),
        out_shape=[
            jax.ShapeDtypeStruct((B, L, D), jnp.float32),
            jax.ShapeDtypeStruct((B, L, D), jnp.float32),
        ],
        compiler_params=pltpu.CompilerParams(
            dimension_semantics=("arbitrary",)),
    )(seq32, tab, stats, x, x, x, x, y, y, y, y)

    mask_f = pl.pallas_call(
        _mask_kernel,
        in_specs=[pl.BlockSpec(memory_space=pltpu.SMEM)],
        out_specs=pl.BlockSpec((B, L), lambda: (0, 0)),
        out_shape=jax.ShapeDtypeStruct((B, L), jnp.float32),
    )(seq32)

    mask = mask_f.astype(bool)
    return (x_out, y_out, seq_len, mask)


# speedup vs baseline: 1.2128x; 1.0040x over previous
import jax
import jax.numpy as jnp
from jax.experimental import pallas as pl
from jax.experimental.pallas import tpu as pltpu

B, L, D = 16, 4096, 256
NQ = 4
TQ = L // NQ
G = TQ // 8
N = B * L


def _q_index_map(q):
    def imap(b, seq_ref, tab_ref):
        return (tab_ref[q, b], q, 0)
    return imap


def _finalize(acc_ref, stats_ref):
    inv_n = 1.0 / N
    inv_nm1 = 1.0 / (N - 1)
    s_x = jnp.sum(acc_ref[0], axis=0, keepdims=True)
    ss_x = jnp.sum(acc_ref[1], axis=0, keepdims=True)
    s_y = jnp.sum(acc_ref[2], axis=0, keepdims=True)
    ss_y = jnp.sum(acc_ref[3], axis=0, keepdims=True)
    x_loc = s_x * inv_n
    y_loc = s_y * inv_n
    x_var = (ss_x - N * x_loc * x_loc) * inv_nm1
    y_var = (ss_y - N * y_loc * y_loc) * inv_nm1
    x_rs = jax.lax.rsqrt(x_var)
    y_rs = jax.lax.rsqrt(y_var)
    rows = jnp.concatenate([x_rs, -x_loc * x_rs, y_rs, -y_loc * y_rs], axis=0)
    stats_ref[...] = jnp.broadcast_to(rows[:, None, :], (4, 8, D))


def _stats_kernel(seq_ref, tab_ref, *refs):
    x_refs = refs[0:NQ]
    y_refs = refs[NQ:2 * NQ]
    stats_ref = refs[2 * NQ]
    acc_ref = refs[2 * NQ + 1]
    b = pl.program_id(0)

    @pl.when(b == 0)
    def _():
        acc_ref[...] = jnp.zeros_like(acc_ref)

    slen = seq_ref[b]
    col0 = jax.lax.broadcasted_iota(jnp.int32, (8, D), 1) == 0

    def accumulate(xm, ym, d):
        acc_ref[0] += jnp.sum(xm, axis=0)
        acc_ref[1] += jnp.sum(xm * xm, axis=0)
        s_y = jnp.sum(ym, axis=0)
        ss_y = jnp.sum(ym * ym, axis=0)
        s_d = jnp.sum(d, axis=0)
        ss_d = jnp.sum(d * d, axis=0)
        acc_ref[2] += jnp.where(col0, s_d, s_y)
        acc_ref[3] += jnp.where(col0, ss_d, ss_y)

    for q in range(NQ):
        start = q * TQ

        @pl.when(start + TQ <= slen)
        def _(q=q):
            xb = x_refs[q][0].reshape(G, 8, D)
            yb = y_refs[q][0].reshape(G, 8, D)
            d = xb[:, :, 0:1] - yb[:, :, 0:1]
            accumulate(xb, yb, d)

        @pl.when((start < slen) & (start + TQ > slen))
        def _(q=q, start=start):
            xb = x_refs[q][0].reshape(G, 8, D)
            yb = y_refs[q][0].reshape(G, 8, D)
            rows = (jax.lax.broadcasted_iota(jnp.int32, (G, 8, 1), 0) * 8
                    + jax.lax.broadcasted_iota(jnp.int32, (G, 8, 1), 1)
                    + start)
            valid = rows < slen
            xm = jnp.where(valid, xb, 0.0)
            ym = jnp.where(valid, yb, 0.0)
            d = xm[:, :, 0:1] - ym[:, :, 0:1]
            accumulate(xm, ym, d)

    @pl.when(b == B - 1)
    def _():
        _finalize(acc_ref, stats_ref)


def _norm_kernel(seq_ref, tab_ref, *refs):
    stats_ref = refs[0]
    x_refs = refs[1:1 + NQ]
    y_refs = refs[1 + NQ:1 + 2 * NQ]
    xo_ref = refs[1 + 2 * NQ]
    yo_ref = refs[2 + 2 * NQ]
    b = pl.program_id(0)
    slen = seq_ref[b]

    x_rs = stats_ref[0]
    x_fill = stats_ref[1]
    y_rs = stats_ref[2]
    y_fill = stats_ref[3]
    y_rs0 = stats_ref[2, :, 0:1]
    y_fill0 = stats_ref[3, :, 0:1]

    for q in range(NQ):
        start = q * TQ
        sl = slice(start, start + TQ)

        @pl.when(start + TQ <= slen)
        def _(q=q, sl=sl):
            xb = x_refs[q][0].reshape(G, 8, D)
            yb = y_refs[q][0].reshape(G, 8, D)
            xo_ref[0, sl, :] = (xb * x_rs + x_fill).reshape(TQ, D)
            yo_ref[0, sl, :] = (yb * y_rs + y_fill).reshape(TQ, D)
            d = xb[:, :, 0:1] - yb[:, :, 0:1]
            yo_ref[0, sl, 0:1] = (d * y_rs0 + y_fill0).reshape(TQ, 1)

        @pl.when((start < slen) & (start + TQ > slen))
        def _(q=q, sl=sl, start=start):
            xb = x_refs[q][0].reshape(G, 8, D)
            yb = y_refs[q][0].reshape(G, 8, D)
            rows = (jax.lax.broadcasted_iota(jnp.int32, (G, 8, 1), 0) * 8
                    + jax.lax.broadcasted_iota(jnp.int32, (G, 8, 1), 1)
                    + start)
            valid = rows < slen
            xo_ref[0, sl, :] = jnp.where(
                valid, xb * x_rs + x_fill, x_fill).reshape(TQ, D)
            yo_ref[0, sl, :] = jnp.where(
                valid, yb * y_rs + y_fill, y_fill).reshape(TQ, D)
            d = xb[:, :, 0:1] - yb[:, :, 0:1]
            yo_ref[0, sl, 0:1] = jnp.where(
                valid, d * y_rs0 + y_fill0, y_fill0).reshape(TQ, 1)

        @pl.when(start >= slen)
        def _(sl=sl):
            xo_ref[0, sl, :] = jnp.broadcast_to(
                x_fill, (G, 8, D)).reshape(TQ, D)
            yo_ref[0, sl, :] = jnp.broadcast_to(
                y_fill, (G, 8, D)).reshape(TQ, D)


def _mask_kernel(seq_ref, m_ref):
    cols = jax.lax.broadcasted_iota(jnp.int32, (1, L), 1)
    for b in range(B):
        m_ref[b:b + 1, :] = (cols < seq_ref[b]).astype(jnp.float32)


def kernel(x, y, seq_len):
    seq32 = seq_len.astype(jnp.int32)

    thresh = (jnp.arange(NQ, dtype=jnp.int32) * TQ)[:, None]
    ok = seq32[None, :] > thresh
    idx = jnp.where(ok, jnp.arange(B, dtype=jnp.int32)[None, :], -1)
    tab = jnp.maximum(jax.lax.cummax(idx, axis=1), 0)

    q_in_specs = [pl.BlockSpec((1, TQ, D), _q_index_map(q)) for q in range(NQ)]

    stats = pl.pallas_call(
        _stats_kernel,
        grid_spec=pltpu.PrefetchScalarGridSpec(
            num_scalar_prefetch=2,
            grid=(B,),
            in_specs=q_in_specs + q_in_specs,
            out_specs=pl.BlockSpec((4, 8, D),
                                   lambda b, seq_ref, tab_ref: (0, 0, 0)),
            scratch_shapes=[pltpu.VMEM((4, 8, D), jnp.float32)],
        ),
        out_shape=jax.ShapeDtypeStruct((4, 8, D), jnp.float32),
        compiler_params=pltpu.CompilerParams(
            dimension_semantics=("arbitrary",)),
    )(seq32, tab, x, x, x, x, y, y, y, y)

    x_out, y_out = pl.pallas_call(
        _norm_kernel,
        grid_spec=pltpu.PrefetchScalarGridSpec(
            num_scalar_prefetch=2,
            grid=(B,),
            in_specs=[pl.BlockSpec((4, 8, D),
                                   lambda b, seq_ref, tab_ref: (0, 0, 0))]
            + q_in_specs + q_in_specs,
            out_specs=[
                pl.BlockSpec((1, L, D), lambda b, seq_ref, tab_ref: (b, 0, 0)),
                pl.BlockSpec((1, L, D), lambda b, seq_ref, tab_ref: (b, 0, 0)),
            ],
        ),
        out_shape=[
            jax.ShapeDtypeStruct((B, L, D), jnp.float32),
            jax.ShapeDtypeStruct((B, L, D), jnp.float32),
        ],
        compiler_params=pltpu.CompilerParams(
            dimension_semantics=("arbitrary",)),
    )(seq32, tab, stats, x, x, x, x, y, y, y, y)

    mask_f = pl.pallas_call(
        _mask_kernel,
        in_specs=[pl.BlockSpec(memory_space=pltpu.SMEM)],
        out_specs=pl.BlockSpec((B, L), lambda: (0, 0)),
        out_shape=jax.ShapeDtypeStruct((B, L), jnp.float32),
    )(seq32)

    mask = mask_f.astype(bool)
    return (x_out, y_out, seq_len, mask)


# speedup vs baseline: 1.2645x; 1.0426x over previous
import jax
import jax.numpy as jnp
from jax.experimental import pallas as pl
from jax.experimental.pallas import tpu as pltpu

B, L, D = 16, 4096, 256
NQ = 4
TQ = L // NQ
G = TQ // 8
N = B * L


def _q_index_map(q):
    def imap(p, seq_ref, tab_ref):
        b = jnp.where(p < B, p, p - B)
        return (tab_ref[q, b], q, 0)
    return imap


def _out_index_map(p, seq_ref, tab_ref):
    return (jnp.maximum(p - B, 0), 0, 0)


def _finalize(acc_ref, stats_ref):
    inv_n = 1.0 / N
    inv_nm1 = 1.0 / (N - 1)
    s_x = jnp.sum(acc_ref[0], axis=0, keepdims=True)
    ss_x = jnp.sum(acc_ref[1], axis=0, keepdims=True)
    s_y = jnp.sum(acc_ref[2], axis=0, keepdims=True)
    ss_y = jnp.sum(acc_ref[3], axis=0, keepdims=True)
    x_loc = s_x * inv_n
    y_loc = s_y * inv_n
    x_var = (ss_x - N * x_loc * x_loc) * inv_nm1
    y_var = (ss_y - N * y_loc * y_loc) * inv_nm1
    x_rs = jax.lax.rsqrt(x_var)
    y_rs = jax.lax.rsqrt(y_var)
    rows = jnp.concatenate([x_rs, -x_loc * x_rs, y_rs, -y_loc * y_rs], axis=0)
    stats_ref[...] = jnp.broadcast_to(rows[:, None, :], (4, 8, D))


def _fused_kernel(seq_ref, tab_ref, *refs):
    x_refs = refs[0:NQ]
    y_refs = refs[NQ:2 * NQ]
    xo_ref = refs[2 * NQ]
    yo_ref = refs[2 * NQ + 1]
    m_ref = refs[2 * NQ + 2]
    acc_ref = refs[2 * NQ + 3]
    stats_ref = refs[2 * NQ + 4]

    p = pl.program_id(0)

    @pl.when(p == 0)
    def _():
        acc_ref[...] = jnp.zeros_like(acc_ref)

    @pl.when(p < B)
    def _():
        slen = seq_ref[p]
        col0 = jax.lax.broadcasted_iota(jnp.int32, (8, D), 1) == 0

        def accumulate(xm, ym, d):
            acc_ref[0] += jnp.sum(xm, axis=0)
            acc_ref[1] += jnp.sum(xm * xm, axis=0)
            s_y = jnp.sum(ym, axis=0)
            ss_y = jnp.sum(ym * ym, axis=0)
            s_d = jnp.sum(d, axis=0)
            ss_d = jnp.sum(d * d, axis=0)
            acc_ref[2] += jnp.where(col0, s_d, s_y)
            acc_ref[3] += jnp.where(col0, ss_d, ss_y)

        for q in range(NQ):
            start = q * TQ

            @pl.when(start + TQ <= slen)
            def _(q=q):
                xb = x_refs[q][0].reshape(G, 8, D)
                yb = y_refs[q][0].reshape(G, 8, D)
                d = xb[:, :, 0:1] - yb[:, :, 0:1]
                accumulate(xb, yb, d)

            @pl.when((start < slen) & (start + TQ > slen))
            def _(q=q, start=start):
                xb = x_refs[q][0].reshape(G, 8, D)
                yb = y_refs[q][0].reshape(G, 8, D)
                rows = (jax.lax.broadcasted_iota(jnp.int32, (G, 8, 1), 0) * 8
                        + jax.lax.broadcasted_iota(jnp.int32, (G, 8, 1), 1)
                        + start)
                valid = rows < slen
                xm = jnp.where(valid, xb, 0.0)
                ym = jnp.where(valid, yb, 0.0)
                d = xm[:, :, 0:1] - ym[:, :, 0:1]
                accumulate(xm, ym, d)

        @pl.when(p == B - 1)
        def _():
            _finalize(acc_ref, stats_ref)

    @pl.when(p >= B)
    def _():
        b = p - B
        slen = seq_ref[b]

        x_rs = stats_ref[0]
        x_fill = stats_ref[1]
        y_rs = stats_ref[2]
        y_fill = stats_ref[3]
        y_rs0 = stats_ref[2, :, 0:1]
        y_fill0 = stats_ref[3, :, 0:1]

        cols = jax.lax.broadcasted_iota(jnp.int32, (1, 1, L), 2)
        m_ref[...] = (cols < slen).astype(jnp.float32)

        for q in range(NQ):
            start = q * TQ
            sl = slice(start, start + TQ)

            @pl.when(start + TQ <= slen)
            def _(q=q, sl=sl):
                xb = x_refs[q][0].reshape(G, 8, D)
                yb = y_refs[q][0].reshape(G, 8, D)
                xo_ref[0, sl, :] = (xb * x_rs + x_fill).reshape(TQ, D)
                yo_ref[0, sl, :] = (yb * y_rs + y_fill).reshape(TQ, D)
                d = xb[:, :, 0:1] - yb[:, :, 0:1]
                yo_ref[0, sl, 0:1] = (d * y_rs0 + y_fill0).reshape(TQ, 1)

            @pl.when((start < slen) & (start + TQ > slen))
            def _(q=q, sl=sl, start=start):
                xb = x_refs[q][0].reshape(G, 8, D)
                yb = y_refs[q][0].reshape(G, 8, D)
                rows = (jax.lax.broadcasted_iota(jnp.int32, (G, 8, 1), 0) * 8
                        + jax.lax.broadcasted_iota(jnp.int32, (G, 8, 1), 1)
                        + start)
                valid = rows < slen
                xo_ref[0, sl, :] = jnp.where(
                    valid, xb * x_rs + x_fill, x_fill).reshape(TQ, D)
                yo_ref[0, sl, :] = jnp.where(
                    valid, yb * y_rs + y_fill, y_fill).reshape(TQ, D)
                d = xb[:, :, 0:1] - yb[:, :, 0:1]
                yo_ref[0, sl, 0:1] = jnp.where(
                    valid, d * y_rs0 + y_fill0, y_fill0).reshape(TQ, 1)

            @pl.when(start >= slen)
            def _(sl=sl):
                xo_ref[0, sl, :] = jnp.broadcast_to(
                    x_fill, (G, 8, D)).reshape(TQ, D)
                yo_ref[0, sl, :] = jnp.broadcast_to(
                    y_fill, (G, 8, D)).reshape(TQ, D)


def kernel(x, y, seq_len):
    seq32 = seq_len.astype(jnp.int32)

    thresh = (jnp.arange(NQ, dtype=jnp.int32) * TQ)[:, None]
    ok = seq32[None, :] > thresh
    idx = jnp.where(ok, jnp.arange(B, dtype=jnp.int32)[None, :], -1)
    tab = jnp.maximum(jax.lax.cummax(idx, axis=1), 0)

    q_in_specs = [pl.BlockSpec((1, TQ, D), _q_index_map(q)) for q in range(NQ)]

    x_out, y_out, mask_f = pl.pallas_call(
        _fused_kernel,
        grid_spec=pltpu.PrefetchScalarGridSpec(
            num_scalar_prefetch=2,
            grid=(2 * B,),
            in_specs=q_in_specs + q_in_specs,
            out_specs=[
                pl.BlockSpec((1, L, D), _out_index_map),
                pl.BlockSpec((1, L, D), _out_index_map),
                pl.BlockSpec((1, 1, L), _out_index_map),
            ],
            scratch_shapes=[pltpu.VMEM((4, 8, D), jnp.float32),
                            pltpu.VMEM((4, 8, D), jnp.float32)],
        ),
        out_shape=[
            jax.ShapeDtypeStruct((B, L, D), jnp.float32),
            jax.ShapeDtypeStruct((B, L, D), jnp.float32),
            jax.ShapeDtypeStruct((B, 1, L), jnp.float32),
        ],
        compiler_params=pltpu.CompilerParams(
            dimension_semantics=("arbitrary",)),
    )(seq32, tab, x, x, x, x, y, y, y, y)

    mask = mask_f.reshape(B, L).astype(bool)
    return (x_out, y_out, seq_len, mask)
